# Initial kernel scaffold; baseline (speedup 1.0000x reference)
import jax
import jax.numpy as jnp
from jax.experimental import pallas as pl


def kernel(iteration):
    raise NotImplementedError("write your pallas kernel here")


# speedup vs baseline: 1.7247x; 1.7247x over previous
import functools

import jax
import jax.numpy as jnp
from jax import lax
from jax.experimental import pallas as pl
from jax.experimental.pallas import tpu as pltpu
from jax.experimental.pallas import tpu_sc as plsc

_B = 16384
_TAU = 5
_NC = 2
_NS = 16
_NW = _NC * _NS
_L = 16
_PER_W = _B // _NW
_OUT_PER_W = 2 * _PER_W

_mesh = plsc.VectorSubcoreMesh(core_axis_name="c", subcore_axis_name="s")


@functools.partial(
    pl.kernel,
    mesh=_mesh,
    out_type=jax.ShapeDtypeStruct((2 * _B,), jnp.float32),
    scratch_types=[
        pltpu.VMEM((_PER_W,), jnp.int32),
        pltpu.VMEM((_OUT_PER_W,), jnp.float32),
    ],
    compiler_params=pltpu.CompilerParams(needs_layout_passes=False),
)
def _router_sc(it_hbm, out_hbm, it_v, out_v):
    wid = lax.axis_index("s") * _NC + lax.axis_index("c")
    base = wid * _PER_W
    pltpu.sync_copy(it_hbm.at[pl.ds(base, _PER_W)], it_v)

    def body(i, carry):
        x = it_v[pl.ds(i * _L, _L)]
        hit = lax.rem(x + 1, _TAU) == 0
        col1 = jnp.where(hit, jnp.float32(1.0), jnp.float32(0.0))
        col0 = jnp.float32(1.0) - col1
        idx = (lax.iota(jnp.int32, _L) + i * _L) * 2
        plsc.store_scatter(out_v, [idx], col0)
        plsc.store_scatter(out_v, [idx + 1], col1)
        return carry

    lax.fori_loop(0, _PER_W // _L, body, 0)
    pltpu.sync_copy(out_v, out_hbm.at[pl.ds(2 * base, _OUT_PER_W)])


def kernel(iteration):
    flat = _router_sc(iteration.astype(jnp.int32))
    return flat.reshape(_B, 2)


# speedup vs baseline: 2.1923x; 1.2711x over previous
import functools

import jax
import jax.numpy as jnp
from jax import lax
from jax.experimental import pallas as pl
from jax.experimental.pallas import tpu as pltpu
from jax.experimental.pallas import tpu_sc as plsc

_B = 16384
_TAU = 5
_NC = 2
_NS = 16
_NW = _NC * _NS
_L = 16
_PER_W = _B // _NW

_mesh = plsc.VectorSubcoreMesh(core_axis_name="c", subcore_axis_name="s")


@functools.partial(
    pl.kernel,
    mesh=_mesh,
    out_type=jax.ShapeDtypeStruct((_B, 2), jnp.float32),
    scratch_types=[
        pltpu.VMEM((_PER_W,), jnp.int32),
        pltpu.VMEM((_PER_W, 2), jnp.float32),
    ],
    compiler_params=pltpu.CompilerParams(needs_layout_passes=False),
)
def _router_sc(it_hbm, out_hbm, it_v, out_v):
    wid = lax.axis_index("s") * _NC + lax.axis_index("c")
    base = wid * _PER_W
    pltpu.sync_copy(it_hbm.at[pl.ds(base, _PER_W)], it_v)

    zeros = jnp.zeros((_L,), jnp.int32)
    ones = zeros + 1

    def body(i, carry):
        x = it_v[pl.ds(i * _L, _L)]
        hit = lax.rem(x + 1, _TAU) == 0
        col1 = jnp.where(hit, jnp.float32(1.0), jnp.float32(0.0))
        col0 = jnp.float32(1.0) - col1
        rows = lax.iota(jnp.int32, _L) + i * _L
        plsc.store_scatter(out_v, [rows, zeros], col0)
        plsc.store_scatter(out_v, [rows, ones], col1)
        return carry

    lax.fori_loop(0, _PER_W // _L, body, 0)
    pltpu.sync_copy(out_v, out_hbm.at[pl.ds(base, _PER_W), :])


def kernel(iteration):
    return _router_sc(iteration.astype(jnp.int32))


# speedup vs baseline: 3.1795x; 1.4503x over previous
import functools

import jax
import jax.numpy as jnp
from jax import lax
from jax.experimental import pallas as pl
from jax.experimental.pallas import tpu as pltpu
from jax.experimental.pallas import tpu_sc as plsc

_B = 16384
_TAU = 5
_NC = 2
_NS = 16
_NW = _NC * _NS
_L = 16
_PER_W = _B // _NW

_mesh = plsc.VectorSubcoreMesh(core_axis_name="c", subcore_axis_name="s")


@functools.partial(
    pl.kernel,
    mesh=_mesh,
    out_type=jax.ShapeDtypeStruct((2, _B), jnp.float32),
    scratch_types=[
        pltpu.VMEM((_PER_W,), jnp.int32),
        pltpu.VMEM((_PER_W,), jnp.float32),
        pltpu.VMEM((_PER_W,), jnp.float32),
    ],
    compiler_params=pltpu.CompilerParams(needs_layout_passes=False),
)
def _router_sc(it_hbm, out_hbm, it_v, c0_v, c1_v):
    wid = lax.axis_index("s") * _NC + lax.axis_index("c")
    base = wid * _PER_W
    pltpu.sync_copy(it_hbm.at[pl.ds(base, _PER_W)], it_v)

    def body(i, carry):
        sl = pl.ds(i * _L, _L)
        x = it_v[sl]
        hit = lax.rem(x + 1, _TAU) == 0
        col1 = jnp.where(hit, jnp.float32(1.0), jnp.float32(0.0))
        c1_v[sl] = col1
        c0_v[sl] = jnp.float32(1.0) - col1
        return carry

    lax.fori_loop(0, _PER_W // _L, body, 0)
    pltpu.sync_copy(c0_v, out_hbm.at[0, pl.ds(base, _PER_W)])
    pltpu.sync_copy(c1_v, out_hbm.at[1, pl.ds(base, _PER_W)])


def kernel(iteration):
    return _router_sc(iteration.astype(jnp.int32)).T
